# baseline (speedup 1.0000x reference)
import functools

import jax
import jax.numpy as jnp
from jax.experimental import pallas as pl
from jax.experimental.pallas import tpu as pltpu

NUM_HEADS = 8
NUM_CODES = 1024
HEAD_DIM = 64
BL = 512


def _pq_kernel(z_ref, cb_ref, zq_ref, idx_ref, loss_ref):
    zb = z_ref[0]
    cb = cb_ref[0]
    sims = jnp.dot(cb, zb, preferred_element_type=jnp.float32)
    idx = jnp.argmax(sims, axis=0).astype(jnp.int32)
    onehot = (jax.lax.broadcasted_iota(jnp.int32, sims.shape, 0)
              == idx[None, :]).astype(jnp.bfloat16)
    cb_hi = cb.astype(jnp.bfloat16)
    cb_lo = (cb - cb_hi.astype(jnp.float32)).astype(jnp.bfloat16)
    zq = (jnp.dot(cb_hi.T, onehot, preferred_element_type=jnp.float32)
          + jnp.dot(cb_lo.T, onehot, preferred_element_type=jnp.float32))
    zq_ref[0] = zq
    idx_ref[0, 0, 0] = idx
    part = jnp.sum((zb - zq) ** 2)

    lb = pl.program_id(2)

    @pl.when(lb == 0)
    def _init():
        loss_ref[0, 0, 0] = jnp.zeros((128,), jnp.float32)

    loss_ref[0, 0, 0] = loss_ref[0, 0, 0] + part


@functools.partial(jax.jit, static_argnames=("interpret",))
def kernel(z, codebooks, interpret=False):
    b, d_model, l = z.shape
    h, c, d = codebooks.shape

    grid = (h, b, l // BL)
    zq, idx, loss_parts = pl.pallas_call(
        _pq_kernel,
        grid=grid,
        in_specs=[
            pl.BlockSpec((1, d, BL), lambda hh, bb, lb: (bb, hh, lb)),
            pl.BlockSpec((1, c, d), lambda hh, bb, lb: (hh, 0, 0)),
        ],
        out_specs=[
            pl.BlockSpec((1, d, BL), lambda hh, bb, lb: (bb, hh, lb)),
            pl.BlockSpec((1, 1, 1, BL), lambda hh, bb, lb: (bb, hh, 0, lb)),
            pl.BlockSpec((1, 1, 1, 128), lambda hh, bb, lb: (hh, bb, 0, 0)),
        ],
        out_shape=[
            jax.ShapeDtypeStruct((b, d_model, l), jnp.float32),
            jax.ShapeDtypeStruct((b, h, 1, l), jnp.int32),
            jax.ShapeDtypeStruct((h, b, 1, 128), jnp.float32),
        ],
        compiler_params=pltpu.CompilerParams(
            dimension_semantics=("parallel", "parallel", "arbitrary"),
        ),
        interpret=interpret,
    )(z, codebooks)

    n_total = h * b * l * d
    vq_loss = 1.25 * jnp.sum(loss_parts[:, :, 0, 0]) / n_total
    return (zq, vq_loss, idx.reshape(b, h, l))


# speedup vs baseline: 1.1813x; 1.1813x over previous
import functools

import jax
import jax.numpy as jnp
from jax.experimental import pallas as pl
from jax.experimental.pallas import tpu as pltpu

NUM_HEADS = 8
NUM_CODES = 1024
HEAD_DIM = 64
BL = 512


def _pq_kernel(z_ref, cb_ref, zq_ref, idx_ref, loss_ref):
    zb = z_ref[0]
    cb = cb_ref[0]
    sims = jnp.dot(cb, zb, preferred_element_type=jnp.float32)
    idx = jnp.argmax(sims, axis=0).astype(jnp.int32)
    onehot = (jax.lax.broadcasted_iota(jnp.int32, sims.shape, 0)
              == idx[None, :]).astype(jnp.float32)
    zq = jnp.dot(cb.T, onehot, preferred_element_type=jnp.float32)
    zq_ref[0] = zq
    idx_ref[0, 0, 0] = idx
    part = jnp.sum((zb - zq) ** 2)

    lb = pl.program_id(2)

    @pl.when(lb == 0)
    def _init():
        loss_ref[0, 0, 0] = jnp.zeros((128,), jnp.float32)

    loss_ref[0, 0, 0] = loss_ref[0, 0, 0] + part


@functools.partial(jax.jit, static_argnames=("interpret",))
def kernel(z, codebooks, interpret=False):
    b, d_model, l = z.shape
    h, c, d = codebooks.shape

    grid = (h, b, l // BL)
    zq, idx, loss_parts = pl.pallas_call(
        _pq_kernel,
        grid=grid,
        in_specs=[
            pl.BlockSpec((1, d, BL), lambda hh, bb, lb: (bb, hh, lb)),
            pl.BlockSpec((1, c, d), lambda hh, bb, lb: (hh, 0, 0)),
        ],
        out_specs=[
            pl.BlockSpec((1, d, BL), lambda hh, bb, lb: (bb, hh, lb)),
            pl.BlockSpec((1, 1, 1, BL), lambda hh, bb, lb: (bb, hh, 0, lb)),
            pl.BlockSpec((1, 1, 1, 128), lambda hh, bb, lb: (hh, bb, 0, 0)),
        ],
        out_shape=[
            jax.ShapeDtypeStruct((b, d_model, l), jnp.float32),
            jax.ShapeDtypeStruct((b, h, 1, l), jnp.int32),
            jax.ShapeDtypeStruct((h, b, 1, 128), jnp.float32),
        ],
        compiler_params=pltpu.CompilerParams(
            dimension_semantics=("parallel", "parallel", "arbitrary"),
        ),
        interpret=interpret,
    )(z, codebooks)

    n_total = h * b * l * d
    vq_loss = 1.25 * jnp.sum(loss_parts[:, :, 0, 0]) / n_total
    return (zq, vq_loss, idx.reshape(b, h, l))
